# Initial kernel scaffold; baseline (speedup 1.0000x reference)
import jax
import jax.numpy as jnp
from jax.experimental import pallas as pl


def kernel(x, edge_index, batch, W1, b1, gamma1, beta1, W2, b2, gamma2, beta2, Wc1, bc1, Wc2, bc2):
    raise NotImplementedError("write your pallas kernel here")


# speedup vs baseline: 9.4862x; 9.4862x over previous
import functools
import jax
import jax.numpy as jnp
from jax import lax
from jax.experimental import pallas as pl
from jax.experimental.pallas import tpu as pltpu
from jax.experimental.pallas import tpu_sc as plsc

N = 10000
E = 160000
NP = 10240
G = 512
GACC = 640
EPAD = 161792
EB = EPAD // 128
EBT = EB // 16
EPT = EPAD // 32
NPT = NP // 32
D1 = 256
D2 = 512
EPS = 1e-5
NB = 512
_F32 = jnp.float32
_I32 = jnp.int32


def _make_deg_kernel():
    mesh = plsc.VectorSubcoreMesh(core_axis_name="c", subcore_axis_name="s", num_cores=2, num_subcores=16)

    @functools.partial(
        pl.kernel,
        out_type=[jax.ShapeDtypeStruct((32, NP), _F32),
                  jax.ShapeDtypeStruct((32, GACC), _F32)],
        mesh=mesh,
        compiler_params=pltpu.CompilerParams(needs_layout_passes=False, use_tc_tiling_on_sc=False),
        scratch_types=[pltpu.VMEM((NP,), _F32),
                       pltpu.VMEM((GACC,), _F32),
                       pltpu.VMEM((EPT,), _I32),
                       pltpu.VMEM((NPT,), _I32)],
    )
    def deg_kernel(dst_hbm, batch_hbm, cnt_out, gcnt_out, hist, ghist, dstv, batchv):
        cid = lax.axis_index("c")
        sid = lax.axis_index("s")
        tid = cid * 16 + sid
        zeros16 = jnp.zeros((16,), _F32)
        ones16 = jnp.ones((16,), _F32)

        def z1(j, _):
            hist[pl.ds(j * 16, 16)] = zeros16
            return 0
        lax.fori_loop(0, NP // 16, z1, 0)

        def z2(j, _):
            ghist[pl.ds(j * 16, 16)] = zeros16
            return 0
        lax.fori_loop(0, GACC // 16, z2, 0)

        pltpu.sync_copy(dst_hbm.at[pl.ds(tid * EPT, EPT)], dstv)
        pltpu.sync_copy(batch_hbm.at[pl.ds(tid * NPT, NPT)], batchv)

        def body(j, _):
            idx = dstv[pl.ds(j * 16, 16)]
            plsc.addupdate_scatter(hist, [idx], ones16)
            return 0
        lax.fori_loop(0, EPT // 16, body, 0)

        def body2(j, _):
            idx = batchv[pl.ds(j * 16, 16)]
            plsc.addupdate_scatter(ghist, [idx], ones16)
            return 0
        lax.fori_loop(0, NPT // 16, body2, 0)

        pltpu.sync_copy(hist, cnt_out.at[tid])
        pltpu.sync_copy(ghist, gcnt_out.at[tid])

    return deg_kernel


def _make_agg_kernel(nch, nout, eb):
    ebt = eb // 16
    nz = nout // 16
    mesh = plsc.VectorSubcoreMesh(core_axis_name="c", subcore_axis_name="s", num_cores=2, num_subcores=16)

    @functools.partial(
        pl.kernel,
        out_type=jax.ShapeDtypeStruct((nch * nout, 128), _F32),
        mesh=mesh,
        compiler_params=pltpu.CompilerParams(needs_layout_passes=False, use_tc_tiling_on_sc=False),
        scratch_types=[pltpu.VMEM((ebt, 128), _I32),
                       pltpu.VMEM((ebt, 128), _I32),
                       pltpu.VMEM((128, 128), _F32),
                       pltpu.VMEM((64, 128), _F32),
                       pltpu.VMEM_SHARED((NP, 128), _F32)],
    )
    def agg_kernel(g_hbm, src_hbm, dst_hbm, out_hbm, srcv, dstv, rows, zbuf, acc):
        cid = lax.axis_index("c")
        sid = lax.axis_index("s")
        zeros16 = jnp.zeros((16,), _F32)

        def zr(r, _):
            def zq(q, _):
                zbuf[r, pl.ds(q * 16, 16)] = zeros16
                return 0
            lax.fori_loop(0, 8, zq, 0)
            return 0
        lax.fori_loop(0, 64, zr, 0)

        def zero_my_slice():
            base = sid * nz
            off = 0
            while off < nz:
                sz = min(64, nz - off)
                pltpu.sync_copy(zbuf.at[pl.ds(0, sz)], acc.at[pl.ds(base + off, sz)])
                off += sz
        zero_my_slice()

        pltpu.sync_copy(dst_hbm.at[pl.ds(sid * ebt, ebt)], dstv)

        for k_i in range(nch // 2):
            cc = cid + 2 * k_i
            plsc.subcore_barrier()
            pltpu.sync_copy(src_hbm.at[pl.ds(cc * eb + sid * ebt, ebt)], srcv)

            def edge_blk(j, _):
                pltpu.sync_copy(g_hbm.at[srcv.at[j]], rows)
                pltpu.sync_copy(rows, acc.at[dstv.at[j]], add=True)
                return 0
            lax.fori_loop(0, ebt, edge_blk, 0)

            plsc.subcore_barrier()
            base = sid * nz
            off = 0
            while off < nz:
                sz = min(128, nz - off)
                pltpu.sync_copy(acc.at[pl.ds(base + off, sz)],
                                out_hbm.at[pl.ds(cc * nout + base + off, sz)])
                off += sz
            if k_i < nch // 2 - 1:
                zero_my_slice()

    return agg_kernel


def _prep_body(cnt_ref, x_ref, g1_ref, dinv_ref):
    i = pl.program_id(0)
    cnt = jnp.sum(cnt_ref[...], axis=1, keepdims=True)
    rid = lax.broadcasted_iota(_I32, (1024, 1), 0) + i * 1024
    deg = cnt + jnp.where(rid < N, 1.0, 0.0)
    dinv = jnp.where(deg > 0, lax.rsqrt(jnp.maximum(deg, 1e-12)), 0.0)
    dinv_ref[...] = dinv
    for c in range(2):
        g1_ref[c] = x_ref[:, c * 128:(c + 1) * 128] * dinv


def _prep_call(cntT, xp):
    grid = (NP // 1024,)
    return pl.pallas_call(
        _prep_body,
        grid=grid,
        in_specs=[pl.BlockSpec((1024, 32), lambda i: (i, 0)),
                  pl.BlockSpec((1024, D1), lambda i: (i, 0))],
        out_specs=[pl.BlockSpec((2, 1024, 128), lambda i: (0, i, 0)),
                   pl.BlockSpec((1024, 1), lambda i: (i, 0))],
        out_shape=[jax.ShapeDtypeStruct((2, NP, 128), _F32),
                   jax.ShapeDtypeStruct((NP, 1), _F32)],
    )(cntT, xp)


def _make_mm_body(nch):
    def body(agg_ref, x_ref, dinv_ref, w_ref, b_ref, h_ref, st_ref):
        i = pl.program_id(0)
        d = dinv_ref[...]
        d2 = d * d
        acc = jnp.zeros((NB, D2), _F32)
        for c in range(nch):
            a = d * agg_ref[c] + d2 * x_ref[:, c * 128:(c + 1) * 128]
            acc = acc + jnp.dot(a, w_ref[c * 128:(c + 1) * 128, :],
                                preferred_element_type=_F32)
        h = acc + b_ref[...]
        h_ref[...] = h
        rid = lax.broadcasted_iota(_I32, (NB, 1), 0) + i * NB
        m = jnp.where(rid < N, 1.0, 0.0)
        hm = h * m
        s1 = jnp.sum(hm, axis=0, keepdims=True)
        s2 = jnp.sum(h * hm, axis=0, keepdims=True)

        @pl.when(i == 0)
        def _():
            st_ref[...] = jnp.zeros((8, D2), _F32)
        st_ref[0:2, :] = st_ref[0:2, :] + jnp.concatenate([s1, s2], axis=0)
    return body


def _mm_call(nch, agg, xin, dinv, W, b):
    grid = (NP // NB,)
    return pl.pallas_call(
        _make_mm_body(nch),
        grid=grid,
        in_specs=[pl.BlockSpec((nch, NB, 128), lambda i: (0, i, 0)),
                  pl.BlockSpec((NB, nch * 128), lambda i: (i, 0)),
                  pl.BlockSpec((NB, 1), lambda i: (i, 0)),
                  pl.BlockSpec((nch * 128, D2), lambda i: (0, 0)),
                  pl.BlockSpec((1, D2), lambda i: (0, 0))],
        out_specs=[pl.BlockSpec((NB, D2), lambda i: (i, 0)),
                   pl.BlockSpec((8, D2), lambda i: (0, 0))],
        out_shape=[jax.ShapeDtypeStruct((NP, D2), _F32),
                   jax.ShapeDtypeStruct((8, D2), _F32)],
    )(agg, xin, dinv, W, b)


def _bn1_body(h_ref, st_ref, gamma_ref, beta_ref, dinv_ref, hout_ref, g2_ref):
    mean = st_ref[0:1, :] / N
    var = st_ref[1:2, :] / N - mean * mean
    inv = lax.rsqrt(var + EPS) * gamma_ref[...]
    h = jnp.maximum((h_ref[...] - mean) * inv + beta_ref[...], 0.0)
    hout_ref[...] = h
    d = dinv_ref[...]
    for c in range(4):
        g2_ref[c] = h[:, c * 128:(c + 1) * 128] * d


def _bn1_call(hpre, st, gamma, beta, dinv):
    grid = (NP // NB,)
    return pl.pallas_call(
        _bn1_body,
        grid=grid,
        in_specs=[pl.BlockSpec((NB, D2), lambda i: (i, 0)),
                  pl.BlockSpec((8, D2), lambda i: (0, 0)),
                  pl.BlockSpec((1, D2), lambda i: (0, 0)),
                  pl.BlockSpec((1, D2), lambda i: (0, 0)),
                  pl.BlockSpec((NB, 1), lambda i: (i, 0))],
        out_specs=[pl.BlockSpec((NB, D2), lambda i: (i, 0)),
                   pl.BlockSpec((4, NB, 128), lambda i: (0, i, 0))],
        out_shape=[jax.ShapeDtypeStruct((NP, D2), _F32),
                   jax.ShapeDtypeStruct((4, NP, 128), _F32)],
    )(hpre, st, gamma, beta, dinv)


def _bn2_body(h_ref, st_ref, gamma_ref, beta_ref, g2_ref):
    mean = st_ref[0:1, :] / N
    var = st_ref[1:2, :] / N - mean * mean
    inv = lax.rsqrt(var + EPS) * gamma_ref[...]
    h = jnp.maximum((h_ref[...] - mean) * inv + beta_ref[...], 0.0)
    for c in range(4):
        g2_ref[c] = h[:, c * 128:(c + 1) * 128]


def _bn2_call(hpre, st, gamma, beta):
    grid = (NP // NB,)
    return pl.pallas_call(
        _bn2_body,
        grid=grid,
        in_specs=[pl.BlockSpec((NB, D2), lambda i: (i, 0)),
                  pl.BlockSpec((8, D2), lambda i: (0, 0)),
                  pl.BlockSpec((1, D2), lambda i: (0, 0)),
                  pl.BlockSpec((1, D2), lambda i: (0, 0))],
        out_specs=pl.BlockSpec((4, NB, 128), lambda i: (0, i, 0)),
        out_shape=jax.ShapeDtypeStruct((4, NP, 128), _F32),
    )(hpre, st, gamma, beta)


def _cls_body(pool_ref, gh_ref, w1_ref, b1_ref, w2_ref, b2_ref, out_ref):
    counts = jnp.sum(gh_ref[...], axis=1, keepdims=True)[0:G, :]
    pooled = jnp.concatenate([pool_ref[c, 0:G, :] for c in range(4)], axis=1)
    pooled = pooled / jnp.maximum(counts, 1.0)
    hid = jnp.maximum(jnp.dot(pooled, w1_ref[...], preferred_element_type=_F32)
                      + b1_ref[...], 0.0)
    out_ref[...] = jnp.dot(hid, w2_ref[...], preferred_element_type=_F32) + b2_ref[...]


def _cls_call(pool_raw, ghT, Wc1, bc1, Wc2, bc2):
    return pl.pallas_call(
        _cls_body,
        out_shape=jax.ShapeDtypeStruct((G, 64), _F32),
    )(pool_raw, ghT, Wc1, bc1, Wc2, bc2)


_sc_cache = {}


def _sc_kernels():
    if not _sc_cache:
        _sc_cache["deg"] = _make_deg_kernel()
        _sc_cache["agg2"] = _make_agg_kernel(2, NP, EB)
        _sc_cache["agg4"] = _make_agg_kernel(4, NP, EB)
        _sc_cache["pool"] = _make_agg_kernel(4, GACC, NP // 128)
    return _sc_cache


def kernel(x, edge_index, batch, W1, b1, gamma1, beta1, W2, b2, gamma2, beta2,
           Wc1, bc1, Wc2, bc2):
    src = edge_index[0].astype(_I32)
    dst = edge_index[1].astype(_I32)
    pad = jnp.full((EPAD - E,), N, _I32)
    srcp = jnp.concatenate([src, pad])
    dstp = jnp.concatenate([dst, pad])
    dst2d = dstp.reshape(EB, 128)
    src2 = (srcp[None, :] + (jnp.arange(2, dtype=_I32) * NP)[:, None]).reshape(2 * EB, 128)
    src4 = (srcp[None, :] + (jnp.arange(4, dtype=_I32) * NP)[:, None]).reshape(4 * EB, 128)
    batchp = jnp.concatenate([batch.astype(_I32), jnp.full((NP - N,), G, _I32)])
    pool_src = (jnp.arange(4 * NP, dtype=_I32)).reshape(4 * (NP // 128), 128)
    batch2d = batchp.reshape(NP // 128, 128)
    xp = jnp.pad(x, ((0, NP - N), (0, 0)))
    sck = _sc_kernels()

    cnt32, gh32 = sck["deg"](dstp, batchp)
    g1, dinv = _prep_call(cnt32.T, xp)

    agg1 = sck["agg2"](g1.reshape(2 * NP, 128), src2, dst2d)
    h1pre, st1 = _mm_call(2, agg1.reshape(2, NP, 128), xp, dinv, W1,
                          b1.reshape(1, D2))
    h1, g2 = _bn1_call(h1pre, st1, gamma1.reshape(1, D2), beta1.reshape(1, D2), dinv)

    agg2 = sck["agg4"](g2.reshape(4 * NP, 128), src4, dst2d)
    h2pre, st2 = _mm_call(4, agg2.reshape(4, NP, 128), h1, dinv, W2,
                          b2.reshape(1, D2))
    h2c = _bn2_call(h2pre, st2, gamma2.reshape(1, D2), beta2.reshape(1, D2))

    pool_raw = sck["pool"](h2c.reshape(4 * NP, 128), pool_src, batch2d)
    out = _cls_call(pool_raw.reshape(4, GACC, 128), gh32.T, Wc1,
                    bc1.reshape(1, D2), Wc2, bc2.reshape(1, 64))
    return out
